# baseline (speedup 1.0000x reference)
import jax
import jax.numpy as jnp
from jax.experimental import pallas as pl

_BS = 256


def _body(codes_ref, x_ref, w_ref, abs_ref, o_ref):
    acc = x_ref[0] + abs_ref[...]
    codes = codes_ref[0]
    for r in range(1, 5):
        mask = (codes == r).astype(jnp.float32)
        acc = acc + mask * w_ref[r : r + 1, :]
    o_ref[0] = acc


def kernel(x, pos_codes, struct_w, abs_emb):
    b, s, d = x.shape
    codes3 = pos_codes.astype(jnp.int32).reshape(b, s, 1)
    n_s = s // _BS
    grid = (n_s, b)
    out = pl.pallas_call(
        _body,
        grid=grid,
        in_specs=[
            pl.BlockSpec((1, _BS, 1), lambda si, bi: (bi, si, 0)),
            pl.BlockSpec((1, _BS, d), lambda si, bi: (bi, si, 0)),
            pl.BlockSpec((5, d), lambda si, bi: (0, 0)),
            pl.BlockSpec((_BS, d), lambda si, bi: (si, 0)),
        ],
        out_specs=pl.BlockSpec((1, _BS, d), lambda si, bi: (bi, si, 0)),
        out_shape=jax.ShapeDtypeStruct((b, s, d), x.dtype),
    )(codes3, x, struct_w, abs_emb)
    return out


# speedup vs baseline: 1.3873x; 1.3873x over previous
import jax
import jax.numpy as jnp
from jax.experimental import pallas as pl

_BS = 1024


def _body(codes_ref, x_ref, w_ref, abs_ref, o_ref):
    acc = x_ref[0] + abs_ref[...]
    codes = codes_ref[0]
    for r in range(1, 5):
        mask = (codes == r).astype(jnp.float32)
        acc = acc + mask * w_ref[r : r + 1, :]
    o_ref[0] = acc


def kernel(x, pos_codes, struct_w, abs_emb):
    b, s, d = x.shape
    codes3 = pos_codes.astype(jnp.int32).reshape(b, s, 1)
    n_s = s // _BS
    grid = (n_s, b)
    out = pl.pallas_call(
        _body,
        grid=grid,
        in_specs=[
            pl.BlockSpec((1, _BS, 1), lambda si, bi: (bi, si, 0)),
            pl.BlockSpec((1, _BS, d), lambda si, bi: (bi, si, 0)),
            pl.BlockSpec((5, d), lambda si, bi: (0, 0)),
            pl.BlockSpec((_BS, d), lambda si, bi: (si, 0)),
        ],
        out_specs=pl.BlockSpec((1, _BS, d), lambda si, bi: (bi, si, 0)),
        out_shape=jax.ShapeDtypeStruct((b, s, d), x.dtype),
    )(codes3, x, struct_w, abs_emb)
    return out


# speedup vs baseline: 1.4316x; 1.0320x over previous
import jax
import jax.numpy as jnp
from jax.experimental import pallas as pl

_BS = 2048


def _body(codes_ref, x_ref, w_ref, abs_ref, o_ref):
    acc = x_ref[0] + abs_ref[...]
    codes = codes_ref[0]
    for r in range(1, 5):
        mask = (codes == r).astype(jnp.float32)
        acc = acc + mask * w_ref[r : r + 1, :]
    o_ref[0] = acc


def kernel(x, pos_codes, struct_w, abs_emb):
    b, s, d = x.shape
    codes3 = pos_codes.astype(jnp.int32).reshape(b, s, 1)
    n_s = s // _BS
    grid = (n_s, b)
    out = pl.pallas_call(
        _body,
        grid=grid,
        in_specs=[
            pl.BlockSpec((1, _BS, 1), lambda si, bi: (bi, si, 0)),
            pl.BlockSpec((1, _BS, d), lambda si, bi: (bi, si, 0)),
            pl.BlockSpec((5, d), lambda si, bi: (0, 0)),
            pl.BlockSpec((_BS, d), lambda si, bi: (si, 0)),
        ],
        out_specs=pl.BlockSpec((1, _BS, d), lambda si, bi: (bi, si, 0)),
        out_shape=jax.ShapeDtypeStruct((b, s, d), x.dtype),
    )(codes3, x, struct_w, abs_emb)
    return out
